# baseline (speedup 1.0000x reference)
import functools

import jax
import jax.numpy as jnp
from jax import lax
from jax.experimental import pallas as pl
from jax.experimental.pallas import tpu as pltpu
from jax.experimental.pallas import tpu_sc as plsc

_NUM_CORES = 2
_NUM_SUBCORES = 16
_NW = _NUM_CORES * _NUM_SUBCORES

_B, _T = 16384, 200
_C, _D = 256, 8
_N = _B * _T
_PER_W = _N // _NW
_CHUNK = 10240
_NCHUNK = _PER_W // _CHUNK


def _make_lookup():
    mesh = plsc.VectorSubcoreMesh(core_axis_name="c", subcore_axis_name="s")

    @functools.partial(
        pl.kernel,
        out_type=jax.ShapeDtypeStruct((_N, _D), jnp.float32),
        mesh=mesh,
        scratch_types=[
            pltpu.VMEM((_CHUNK,), jnp.int32),
            pltpu.VMEM((_CHUNK, _D), jnp.float32),
            pltpu.SemaphoreType.DMA,
        ],
        compiler_params=pltpu.CompilerParams(use_tc_tiling_on_sc=False),
    )
    def lookup(idx_hbm, table_hbm, out_hbm, idx_v, rows_v, sem):
        wid = lax.axis_index("s") * _NUM_CORES + lax.axis_index("c")
        for c in range(_NCHUNK):
            base = wid * _PER_W + c * _CHUNK
            pltpu.sync_copy(idx_hbm.at[pl.ds(base, _CHUNK)], idx_v)
            pltpu.async_copy(table_hbm.at[idx_v], rows_v, sem).wait()
            pltpu.sync_copy(rows_v, out_hbm.at[pl.ds(base, _CHUNK)])

    return lookup


_lookup = _make_lookup()


def kernel(idx, codebook):
    b, t = idx.shape
    _, d = codebook.shape
    out = _lookup(idx.reshape(-1), codebook)
    return out.reshape(b, t, d)


# speedup vs baseline: 1.5095x; 1.5095x over previous
import functools

import jax
import jax.numpy as jnp
from jax import lax
from jax.experimental import pallas as pl
from jax.experimental.pallas import tpu as pltpu
from jax.experimental.pallas import tpu_sc as plsc

_NUM_CORES = 2
_NUM_SUBCORES = 16
_NW = _NUM_CORES * _NUM_SUBCORES
_LANES = 16

_B, _T = 16384, 200
_C, _D = 256, 8
_N = _B * _T
_PER_W = _N // _NW
_CHUNK = 6400
_NCHUNK = _PER_W // _CHUNK
_GROUPS = _CHUNK // _LANES


def _make_lookup():
    mesh = plsc.VectorSubcoreMesh(core_axis_name="c", subcore_axis_name="s")

    @functools.partial(
        pl.kernel,
        out_type=jax.ShapeDtypeStruct((_N * _D,), jnp.float32),
        mesh=mesh,
        scratch_types=[
            pltpu.VMEM((_C * _D,), jnp.float32),
            pltpu.VMEM((_CHUNK,), jnp.int32),
            pltpu.VMEM((_CHUNK,), jnp.int32),
            pltpu.VMEM((_CHUNK * _D,), jnp.float32),
            pltpu.VMEM((_CHUNK * _D,), jnp.float32),
            pltpu.SemaphoreType.DMA,
            pltpu.SemaphoreType.DMA,
            pltpu.SemaphoreType.DMA,
            pltpu.SemaphoreType.DMA,
        ],
        compiler_params=pltpu.CompilerParams(needs_layout_passes=False),
    )
    def lookup(idx_hbm, table_hbm, out_hbm, table_v, idx_v0, idx_v1,
               out_v0, out_v1, si0, si1, so0, so1):
        wid = lax.axis_index("s") * _NUM_CORES + lax.axis_index("c")
        pltpu.sync_copy(table_hbm, table_v)
        pos0 = lax.iota(jnp.int32, _LANES) * _D
        idx_bufs, out_bufs = (idx_v0, idx_v1), (out_v0, out_v1)
        isems, osems = (si0, si1), (so0, so1)
        w0 = wid * _PER_W

        def start_idx(c):
            return pltpu.async_copy(
                idx_hbm.at[pl.ds(w0 + c * _CHUNK, _CHUNK)],
                idx_bufs[c % 2], isems[c % 2])

        d_idx, d_out = {}, {}
        d_idx[0] = start_idx(0)
        for c in range(_NCHUNK):
            p = c % 2
            if c + 1 < _NCHUNK:
                d_idx[c + 1] = start_idx(c + 1)
            d_idx[c].wait()
            if c >= 2:
                d_out[c - 2].wait()

            @functools.partial(
                plsc.parallel_loop, 0, _GROUPS, unroll=4)
            def _group(g, idx_v=idx_bufs[p], out_v=out_bufs[p]):
                i16 = idx_v[pl.ds(g * _LANES, _LANES)]
                gidx0 = i16 * _D
                obase = pos0 + g * (_LANES * _D)
                for j in range(_D):
                    vals = plsc.load_gather(table_v, [gidx0 + j])
                    plsc.store_scatter(out_v, [obase + j], vals)

            d_out[c] = pltpu.async_copy(
                out_bufs[p],
                out_hbm.at[pl.ds((w0 + c * _CHUNK) * _D, _CHUNK * _D)],
                osems[p])
        d_out[_NCHUNK - 2].wait()
        d_out[_NCHUNK - 1].wait()

    return lookup


_lookup = _make_lookup()


def kernel(idx, codebook):
    b, t = idx.shape
    _, d = codebook.shape
    out = _lookup(idx.reshape(-1), codebook.reshape(-1))
    return out.reshape(b, t, d)
